# Initial kernel scaffold; baseline (speedup 1.0000x reference)
import jax
import jax.numpy as jnp
from jax.experimental import pallas as pl


def kernel(x, edge_index, edge_type, W, root, bias):
    raise NotImplementedError("write your pallas kernel here")


# speedup vs baseline: 15.7196x; 15.7196x over previous
import functools

import jax
import jax.numpy as jnp
from jax import lax
from jax.experimental import pallas as pl
from jax.experimental.pallas import tpu as pltpu
from jax.experimental.pallas import tpu_sc as plsc

_NC = 2
_NS = 16
_L = 16

_SHOT = 80
_NSHOT = 5
_SUP = _SHOT * _NSHOT


def _xw_matmul(x, W):
    N, D = x.shape
    R = W.shape[0]
    BN = 1000
    nb = N // BN

    def body(x_ref, w_ref, out_ref):
        out_ref[0] = jnp.dot(
            x_ref[...], w_ref[0],
            preferred_element_type=jnp.float32,
            precision=lax.Precision.HIGHEST,
        )

    return pl.pallas_call(
        body,
        grid=(R, nb),
        in_specs=[
            pl.BlockSpec((BN, D), lambda r, i: (i, 0)),
            pl.BlockSpec((1, D, D), lambda r, i: (r, 0, 0)),
        ],
        out_specs=pl.BlockSpec((1, BN, D), lambda r, i: (r, i, 0)),
        out_shape=jax.ShapeDtypeStruct((R, N, D), jnp.float32),
    )(x, W)


def _final(p0, p1, x, root, bias2d):
    N, D = x.shape
    BN = 1000
    nb = N // BN

    def body(p0_ref, p1_ref, x_ref, root_ref, b_ref, out_ref):
        acc = p0_ref[...] + p1_ref[...]
        acc = acc + jnp.dot(
            x_ref[...], root_ref[...],
            preferred_element_type=jnp.float32,
            precision=lax.Precision.HIGHEST,
        )
        acc = acc + b_ref[...]
        out_ref[...] = jnp.maximum(acc, 0.0)

    return pl.pallas_call(
        body,
        grid=(nb,),
        in_specs=[
            pl.BlockSpec((BN, D), lambda i: (i, 0)),
            pl.BlockSpec((BN, D), lambda i: (i, 0)),
            pl.BlockSpec((BN, D), lambda i: (i, 0)),
            pl.BlockSpec((D, D), lambda i: (0, 0)),
            pl.BlockSpec((1, D), lambda i: (0, 0)),
        ],
        out_specs=pl.BlockSpec((BN, D), lambda i: (i, 0)),
        out_shape=jax.ShapeDtypeStruct((N, D), jnp.float32),
    )(p0, p1, x, root, bias2d)


@functools.lru_cache(maxsize=None)
def _make_sc_agg(N, D, R, E):
    per_core = E // _NC
    per_tile = per_core // _NS
    nb_b = per_tile // _SUP
    cnt_per_tile = E // _NS
    nb_a = cnt_per_tile // _SUP
    rows_tile = 640
    n_dump = rows_tile // _SHOT

    mesh = plsc.VectorSubcoreMesh(core_axis_name="c", subcore_axis_name="s")

    @functools.partial(
        pl.kernel,
        out_type=jax.ShapeDtypeStruct((_NC, N, D), jnp.float32),
        mesh=mesh,
        scratch_types=[
            pltpu.VMEM((_SUP,), jnp.int32),
            pltpu.VMEM((_SUP,), jnp.int32),
            pltpu.VMEM((_SUP,), jnp.int32),
            pltpu.VMEM((_NSHOT, _SHOT), jnp.int32),
            pltpu.VMEM((_NSHOT, _SHOT), jnp.int32),
            pltpu.VMEM((_NSHOT, _SHOT), jnp.int32),
            pltpu.VMEM((_SUP,), jnp.float32),
            pltpu.VMEM((_SHOT, D), jnp.float32),
            pltpu.VMEM((_SHOT, D), jnp.float32),
            pltpu.VMEM_SHARED((R * N,), jnp.float32),
            pltpu.VMEM_SHARED((N, D), jnp.float32),
            pltpu.SemaphoreType.DMA,
            pltpu.SemaphoreType.DMA,
        ],
    )
    def sc_agg(xw_hbm, src_hbm, dst_hbm, et_hbm, out_hbm,
               srcv, dstv, etv, rowv, segv, dstv2, fbuf, msga, msgb,
               cnt_sh, acc_sh, sem1, sem2):
        c = lax.axis_index("c")
        s = lax.axis_index("s")

        @pl.loop(0, _SUP // _L)
        def _zf(i):
            fbuf[pl.ds(i * _L, _L)] = jnp.zeros((_L,), jnp.float32)

        cnt_tile = (R * N) // _NS
        @pl.loop(0, cnt_tile // _SUP)
        def _zc(i):
            pltpu.sync_copy(fbuf, cnt_sh.at[pl.ds(s * cnt_tile + i * _SUP, _SUP)])

        @pl.loop(0, (_SHOT * D) // _L)
        def _zm(i):
            r = i // (D // _L)
            col = (i % (D // _L)) * _L
            msga[r, pl.ds(col, _L)] = jnp.zeros((_L,), jnp.float32)

        @pl.loop(0, n_dump)
        def _za(t):
            r0 = s * rows_tile + t * _SHOT
            @pl.when(r0 < N)
            def _():
                pltpu.sync_copy(msga, acc_sh.at[pl.ds(r0, _SHOT)])

        @pl.loop(0, _SUP // _L)
        def _of(i):
            fbuf[pl.ds(i * _L, _L)] = jnp.ones((_L,), jnp.float32)

        plsc.subcore_barrier()

        @pl.loop(0, nb_a)
        def _count(b):
            base = s * cnt_per_tile + b * _SUP
            pltpu.sync_copy(dst_hbm.at[pl.ds(base, _SUP)], dstv)
            pltpu.sync_copy(et_hbm.at[pl.ds(base, _SUP)], etv)

            @pl.loop(0, _SUP // _L)
            def _seg(g):
                sl = pl.ds(g * _L, _L)
                seg = etv[sl] * N + dstv[sl]
                segv[g // _NSHOT, pl.ds((g % _NSHOT) * _L, _L)] = seg

            cps = [pltpu.async_copy(fbuf.at[pl.ds(j * _SHOT, _SHOT)],
                                    cnt_sh.at[segv.at[j]], sem2, add=True)
                   for j in range(_NSHOT)]
            for cp in cps:
                cp.wait()

        plsc.subcore_barrier()

        @pl.loop(0, nb_b)
        def _agg(b):
            base = c * per_core + s * per_tile + b * _SUP
            pltpu.sync_copy(src_hbm.at[pl.ds(base, _SUP)], srcv)
            pltpu.sync_copy(dst_hbm.at[pl.ds(base, _SUP)], dstv)
            pltpu.sync_copy(et_hbm.at[pl.ds(base, _SUP)], etv)

            @pl.loop(0, _SUP // _L)
            def _idx(g):
                sl = pl.ds(g * _L, _L)
                e16 = etv[sl]
                off = pl.ds((g % _NSHOT) * _L, _L)
                rowv[g // _NSHOT, off] = e16 * N + srcv[sl]
                segv[g // _NSHOT, off] = e16 * N + dstv[sl]
                dstv2[g // _NSHOT, off] = dstv[sl]

            cnt_cps = [pltpu.async_copy(cnt_sh.at[segv.at[j]],
                                        fbuf.at[pl.ds(j * _SHOT, _SHOT)], sem2)
                       for j in range(_NSHOT)]
            bufs = [msga if j % 2 == 0 else msgb for j in range(_NSHOT)]
            row_cps = [pltpu.async_copy(xw_hbm.at[rowv.at[0]], bufs[0], sem1)]
            for j in range(_NSHOT):
                if j + 1 < _NSHOT:
                    row_cps.append(
                        pltpu.async_copy(xw_hbm.at[rowv.at[j + 1]],
                                         bufs[j + 1], sem1))
                row_cps[j].wait()
                cnt_cps[j].wait()
                buf = bufs[j]

                @pl.loop(0, _SHOT // _L)
                def _scale(g):
                    sl = pl.ds(j * _SHOT + g * _L, _L)
                    inv = 1.0 / jnp.maximum(fbuf[sl], 1.0)
                    for k in range(_L):
                        sv = lax.gather(
                            inv, jnp.full((_L, 1), k, jnp.int32),
                            lax.GatherDimensionNumbers(
                                offset_dims=(), collapsed_slice_dims=(0,),
                                start_index_map=(0,)),
                            slice_sizes=(1,),
                            mode=lax.GatherScatterMode.PROMISE_IN_BOUNDS)
                        e = g * _L + k
                        for jj in range(D // _L):
                            msl = pl.ds(jj * _L, _L)
                            buf[e, msl] = buf[e, msl] * sv

                pltpu.sync_copy(buf, acc_sh.at[dstv2.at[j]], add=True)

        plsc.subcore_barrier()

        @pl.loop(0, n_dump)
        def _dump(t):
            r0 = s * rows_tile + t * _SHOT
            @pl.when(r0 < N)
            def _():
                pltpu.sync_copy(acc_sh.at[pl.ds(r0, _SHOT)], msga)
                pltpu.sync_copy(msga, out_hbm.at[c, pl.ds(r0, _SHOT)])

    return sc_agg


def kernel(x, edge_index, edge_type, W, root, bias):
    N, D = x.shape
    R = W.shape[0]
    E = edge_type.shape[0]

    src = edge_index[0].astype(jnp.int32)
    dst = edge_index[1].astype(jnp.int32)
    et = edge_type.astype(jnp.int32)

    xw = _xw_matmul(x, W)
    xw_flat = xw.reshape(R * N, D)

    sc_agg = _make_sc_agg(N, D, R, E)
    partial = sc_agg(xw_flat, src, dst, et)

    return _final(partial[0], partial[1], x, root, bias.reshape(1, D))


# speedup vs baseline: 21.8882x; 1.3924x over previous
import functools

import jax
import jax.numpy as jnp
from jax import lax
from jax.experimental import pallas as pl
from jax.experimental.pallas import tpu as pltpu
from jax.experimental.pallas import tpu_sc as plsc

_NC = 2
_NS = 16
_L = 16

_SHOT = 80
_NSHOT = 25
_MB = _SHOT * _NSHOT
_NBUF = 3


def _xw_matmul(x, W):
    N, D = x.shape
    R = W.shape[0]
    BN = 1000
    nb = N // BN

    def body(x_ref, w_ref, out_ref):
        out_ref[0] = jnp.dot(
            x_ref[...], w_ref[0],
            preferred_element_type=jnp.float32,
            precision=lax.Precision.HIGHEST,
        )

    return pl.pallas_call(
        body,
        grid=(R, nb),
        in_specs=[
            pl.BlockSpec((BN, D), lambda r, i: (i, 0)),
            pl.BlockSpec((1, D, D), lambda r, i: (r, 0, 0)),
        ],
        out_specs=pl.BlockSpec((1, BN, D), lambda r, i: (r, i, 0)),
        out_shape=jax.ShapeDtypeStruct((R, N, D), jnp.float32),
    )(x, W)


def _edge_indices(src2, dst2, et2, N):
    rows, cols = src2.shape
    BR = rows
    nb = rows // BR

    def body(s_ref, d_ref, e_ref, row_ref, seg_ref):
        e = e_ref[...]
        row_ref[...] = e * N + s_ref[...]
        seg_ref[...] = e * N + d_ref[...]

    return pl.pallas_call(
        body,
        grid=(nb,),
        in_specs=[pl.BlockSpec((BR, cols), lambda i: (i, 0))] * 3,
        out_specs=[pl.BlockSpec((BR, cols), lambda i: (i, 0))] * 2,
        out_shape=[jax.ShapeDtypeStruct((rows, cols), jnp.int32)] * 2,
    )(src2, dst2, et2)


def _final(p0, p1, x, root, bias2d):
    N, D = x.shape
    BN = 1000
    nb = N // BN

    def body(p0_ref, p1_ref, x_ref, root_ref, b_ref, out_ref):
        acc = p0_ref[...] + p1_ref[...]
        acc = acc + jnp.dot(
            x_ref[...], root_ref[...],
            preferred_element_type=jnp.float32,
            precision=lax.Precision.HIGHEST,
        )
        acc = acc + b_ref[...]
        out_ref[...] = jnp.maximum(acc, 0.0)

    return pl.pallas_call(
        body,
        grid=(nb,),
        in_specs=[
            pl.BlockSpec((BN, D), lambda i: (i, 0)),
            pl.BlockSpec((BN, D), lambda i: (i, 0)),
            pl.BlockSpec((BN, D), lambda i: (i, 0)),
            pl.BlockSpec((D, D), lambda i: (0, 0)),
            pl.BlockSpec((1, D), lambda i: (0, 0)),
        ],
        out_specs=pl.BlockSpec((BN, D), lambda i: (i, 0)),
        out_shape=jax.ShapeDtypeStruct((N, D), jnp.float32),
    )(p0, p1, x, root, bias2d)


@functools.lru_cache(maxsize=None)
def _make_sc_agg(N, D, R, E):
    per_core = E // _NC
    per_tile = per_core // _NS
    nmb_b = per_tile // _MB
    cnt_per_tile = E // _NS
    nmb_a = cnt_per_tile // _MB
    rows_tile = 640
    n_dump = rows_tile // _SHOT
    groups = _MB // _L

    mesh = plsc.VectorSubcoreMesh(core_axis_name="c", subcore_axis_name="s")

    @functools.partial(
        pl.kernel,
        out_type=jax.ShapeDtypeStruct((_NC, N, D), jnp.float32),
        mesh=mesh,
        scratch_types=[
            pltpu.VMEM((_NSHOT, _SHOT), jnp.int32),
            pltpu.VMEM((_NSHOT, _SHOT), jnp.int32),
            pltpu.VMEM((_NSHOT, _SHOT), jnp.int32),
            pltpu.VMEM((_MB,), jnp.float32),
            pltpu.VMEM((_SHOT,), jnp.float32),
            pltpu.VMEM((_SHOT, D), jnp.float32),
            pltpu.VMEM((_SHOT, D), jnp.float32),
            pltpu.VMEM((_SHOT, D), jnp.float32),
            pltpu.VMEM_SHARED((R * N,), jnp.float32),
            pltpu.VMEM_SHARED((N, D), jnp.float32),
            pltpu.SemaphoreType.DMA,
            pltpu.SemaphoreType.DMA,
        ],
    )
    def sc_agg(xw_hbm, row_hbm, seg_hbm, dst_hbm, out_hbm,
               row2, seg2, dst2, cntbuf, ones,
               msg0, msg1, msg2, cnt_sh, acc_sh, sem1, sem2):
        c = lax.axis_index("c")
        s = lax.axis_index("s")
        bufs = [msg0, msg1, msg2]

        def load2d(hbm, base, dst_ref, sem):
            return [pltpu.async_copy(hbm.at[pl.ds(base + j * _SHOT, _SHOT)],
                                     dst_ref.at[j], sem)
                    for j in range(_NSHOT)]

        @pl.loop(0, _MB // _L)
        def _zf(i):
            cntbuf[pl.ds(i * _L, _L)] = jnp.zeros((_L,), jnp.float32)

        cnt_tile = (R * N) // _NS
        @pl.loop(0, cnt_tile // _MB)
        def _zc(i):
            pltpu.sync_copy(cntbuf,
                            cnt_sh.at[pl.ds(s * cnt_tile + i * _MB, _MB)])

        @pl.loop(0, (_SHOT * D) // _L)
        def _zm(i):
            r = i // (D // _L)
            col = (i % (D // _L)) * _L
            msg0[r, pl.ds(col, _L)] = jnp.zeros((_L,), jnp.float32)

        @pl.loop(0, n_dump)
        def _za(t):
            r0 = s * rows_tile + t * _SHOT
            @pl.when(r0 < N)
            def _():
                pltpu.sync_copy(msg0, acc_sh.at[pl.ds(r0, _SHOT)])

        @pl.loop(0, _SHOT // _L)
        def _of(i):
            ones[pl.ds(i * _L, _L)] = jnp.ones((_L,), jnp.float32)

        plsc.subcore_barrier()

        @pl.loop(0, nmb_a)
        def _count(m):
            base = s * cnt_per_tile + m * _MB
            for cp in load2d(seg_hbm, base, seg2, sem2):
                cp.wait()

            cps = [pltpu.async_copy(ones, cnt_sh.at[seg2.at[j]], sem2,
                                    add=True)
                   for j in range(_NSHOT)]
            for cp in cps:
                cp.wait()

        plsc.subcore_barrier()

        def scale_shot(buf, k):
            @pl.loop(0, _SHOT // _L)
            def _scale(g):
                sl = pl.ds(k * _SHOT + g * _L, _L)
                inv = 1.0 / jnp.maximum(cntbuf[sl], 1.0)
                for kk in range(_L):
                    sv = lax.gather(
                        inv, jnp.full((_L, 1), kk, jnp.int32),
                        lax.GatherDimensionNumbers(
                            offset_dims=(), collapsed_slice_dims=(0,),
                            start_index_map=(0,)),
                        slice_sizes=(1,),
                        mode=lax.GatherScatterMode.PROMISE_IN_BOUNDS)
                    e = g * _L + kk
                    for jj in range(D // _L):
                        msl = pl.ds(jj * _L, _L)
                        buf[e, msl] = buf[e, msl] * sv

        def fire_gather(k, i):
            return pltpu.async_copy(
                xw_hbm.at[row2.at[k]], bufs[i], sem1)

        @pl.loop(0, nmb_b)
        def _agg(m):
            base = c * per_core + s * per_tile + m * _MB
            idx_cps = (load2d(dst_hbm, base, dst2, sem2)
                       + load2d(seg_hbm, base, seg2, sem2)
                       + load2d(row_hbm, base, row2, sem2))
            for cp in idx_cps:
                cp.wait()

            cnt_cps = [pltpu.async_copy(
                cnt_sh.at[seg2.at[j]],
                cntbuf.at[pl.ds(j * _SHOT, _SHOT)], sem2)
                for j in range(_NSHOT)]
            for cp in cnt_cps:
                cp.wait()

            for i in range(_NBUF):
                fire_gather(i, i)

            nloop = (_NSHOT - 1) // _NBUF

            @pl.loop(0, nloop)
            def _shots(t):
                k0 = t * _NBUF
                for i in range(_NBUF):
                    k = k0 + i
                    pltpu.make_async_copy(
                        xw_hbm.at[row2.at[k]], bufs[i], sem1).wait()
                    scale_shot(bufs[i], k)
                    pltpu.sync_copy(bufs[i], acc_sh.at[dst2.at[k]], add=True)

                    @pl.when(k + _NBUF < _NSHOT)
                    def _():
                        pltpu.async_copy(
                            xw_hbm.at[row2.at[k + _NBUF]], bufs[i], sem1)

            k_tail = nloop * _NBUF
            pltpu.make_async_copy(
                xw_hbm.at[row2.at[k_tail]], bufs[0], sem1).wait()
            scale_shot(bufs[0], k_tail)
            pltpu.sync_copy(bufs[0], acc_sh.at[dst2.at[k_tail]], add=True)

        plsc.subcore_barrier()

        @pl.loop(0, n_dump)
        def _dump(t):
            r0 = s * rows_tile + t * _SHOT
            @pl.when(r0 < N)
            def _():
                pltpu.sync_copy(acc_sh.at[pl.ds(r0, _SHOT)], msg0)
                pltpu.sync_copy(msg0, out_hbm.at[c, pl.ds(r0, _SHOT)])

    return sc_agg


def kernel(x, edge_index, edge_type, W, root, bias):
    N, D = x.shape
    R = W.shape[0]
    E = edge_type.shape[0]

    src = edge_index[0].astype(jnp.int32)
    dst = edge_index[1].astype(jnp.int32)
    et = edge_type.astype(jnp.int32)

    xw = _xw_matmul(x, W)
    xw_flat = xw.reshape(R * N, D)

    cols = 128
    rows = E // cols
    row2, seg2 = _edge_indices(src.reshape(rows, cols),
                               dst.reshape(rows, cols),
                               et.reshape(rows, cols), N)

    sc_agg = _make_sc_agg(N, D, R, E)
    partial = sc_agg(xw_flat, row2.reshape(E), seg2.reshape(E), dst)

    return _final(partial[0], partial[1], x, root, bias.reshape(1, D))
